# Initial kernel scaffold; baseline (speedup 1.0000x reference)
import jax
import jax.numpy as jnp
from jax.experimental import pallas as pl
from jax.experimental.pallas import tpu as pltpu

N_USERS = 5000
N_ITEMS = 5000
N_NODES = N_USERS + N_ITEMS
EMB = 64
REV = 64


def _dense_body(zu_ref, zi_ref, cu_ref, cii_ref, wint_ref, wrev_ref,
                fuw, fub, fiw, fib,
                fucw, fucb, ficw, ficb,
                fudw, fudb, fidw, fidb,
                furw, furb, firw, firb,
                piw1, piw2, prw1, prw2, pcw1, pcw2, pdw1, pdw2,
                pjw1, pjw2, out_ref):
    f32 = jnp.float32

    def mm(a, b):
        return jnp.dot(a, b, preferred_element_type=f32)

    def side(z, c, fw, fb, fcw, fcb, fdw, fdb, frw, frb):
        s = z[:, 960:1024]
        pieces = []
        rev_pieces = []
        for r in range(5):
            pieces.append(z[:, 64 * r:64 * (r + 1)])
            pieces.append(mm(s, wint_ref[r]))
            rev_pieces.append(mm(s, wrev_ref[r]))
        fall = jnp.concatenate(pieces, axis=1) * c
        fid = mm(fall, fw[...]) + fb[...]
        fcom = mm(z[:, 320:640] * c, fcw[...]) + fcb[...]
        fdis = mm(z[:, 640:960] * c, fdw[...]) + fdb[...]
        frev = mm(jnp.concatenate(rev_pieces, axis=1) * c, frw[...]) + frb[...]
        return fid, fcom, fdis, frev

    zu = zu_ref[...]
    zi = zi_ref[...]
    cu = cu_ref[...]
    cii = cii_ref[...]
    fid_u, fc_u, fd_u, fr_u = side(zu, cu, fuw, fub, fucw, fucb, fudw, fudb,
                                   furw, furb)
    fid_i, fc_i, fd_i, fr_i = side(zi, cii, fiw, fib, ficw, ficb, fidw, fidb,
                                   firw, firb)

    def head(a, b, w1, w2):
        z = a * b
        return mm(jax.nn.relu(mm(z, w1[...])), w2[...])

    oi = head(fid_u, fid_i, piw1, piw2)
    orv = head(fr_u, fr_i, prw1, prw2)
    oc = head(fc_u, fc_i, pcw1, pcw2)
    od = head(fd_u, fd_i, pdw1, pdw2)
    sim = jnp.sum(mm(fc_u, pjw1[...]) * mm(fr_u, pjw2[...]), axis=1,
                  keepdims=True)
    out_ref[...] = jnp.concatenate([oi, orv, oc, od, sim], axis=1)


def _dense_block(zu, zi, cu, cii, wint, wrev, fcs, preds, proj):
    B = zu.shape[0]
    args = [zu, zi, cu, cii, wint, wrev] + fcs + preds + proj
    return pl.pallas_call(
        _dense_body,
        out_shape=jax.ShapeDtypeStruct((B, 21), jnp.float32),
    )(*args)


def kernel(edge_index, users, items, ci, review_feat, weight, weight_com,
           weight_dis, review_w_int, review_w_rev,
           fc_user_w, fc_user_b, fc_item_w, fc_item_b,
           fc_user_com_w, fc_user_com_b, fc_item_com_w, fc_item_com_b,
           fc_user_dis_w, fc_user_dis_b, fc_item_dis_w, fc_item_dis_b,
           fc_user_rev_w, fc_user_rev_b, fc_item_rev_w, fc_item_rev_b,
           pred_int_w1, pred_int_w2, pred_rev_w1, pred_rev_w2,
           pred_com_w1, pred_com_w2, pred_dis_w1, pred_dis_w2,
           proj_w1, proj_w2):
    src, dst = edge_index[0], edge_index[1]
    iid = N_USERS + items

    U = jnp.concatenate([weight, weight_com, weight_dis], axis=0)
    U = (U * ci[None, :, :]).transpose(1, 0, 2).reshape(N_NODES, 15 * EMB)

    g = ci[src]
    S = jax.ops.segment_sum(review_feat * g, dst, num_segments=N_NODES)
    Y = jax.ops.segment_sum(U[src], dst, num_segments=N_NODES)
    Z = jnp.concatenate([Y, S], axis=1)
    zu = Z[users]
    zi = Z[iid]
    cu = ci[users]
    cii = ci[iid]

    fcs = [fc_user_w, fc_user_b.reshape(1, -1), fc_item_w,
           fc_item_b.reshape(1, -1),
           fc_user_com_w, fc_user_com_b.reshape(1, -1), fc_item_com_w,
           fc_item_com_b.reshape(1, -1),
           fc_user_dis_w, fc_user_dis_b.reshape(1, -1), fc_item_dis_w,
           fc_item_dis_b.reshape(1, -1),
           fc_user_rev_w, fc_user_rev_b.reshape(1, -1), fc_item_rev_w,
           fc_item_rev_b.reshape(1, -1)]
    preds = [pred_int_w1, pred_int_w2, pred_rev_w1, pred_rev_w2,
             pred_com_w1, pred_com_w2, pred_dis_w1, pred_dis_w2]
    proj = [proj_w1, proj_w2]
    return _dense_block(zu, zi, cu, cii, review_w_int, review_w_rev, fcs,
                        preds, proj)


# speedup vs baseline: 3.3030x; 3.3030x over previous
import jax
import jax.numpy as jnp
from jax.experimental import pallas as pl
from jax.experimental.pallas import tpu as pltpu

N_USERS = 5000
N_ITEMS = 5000
N_NODES = N_USERS + N_ITEMS
EMB = 64
REV = 64


def _dense_body(zu_ref, zi_ref, cu_ref, cii_ref, wint_ref, wrev_ref,
                fuw, fub, fiw, fib,
                fucw, fucb, ficw, ficb,
                fudw, fudb, fidw, fidb,
                furw, furb, firw, firb,
                piw1, piw2, prw1, prw2, pcw1, pcw2, pdw1, pdw2,
                pjw1, pjw2, out_ref):
    f32 = jnp.float32

    def mm(a, b):
        return jnp.dot(a, b, preferred_element_type=f32)

    def side(z, c, fw, fb, fcw, fcb, fdw, fdb, frw, frb):
        s = z[:, 960:1024]
        pieces = []
        rev_pieces = []
        for r in range(5):
            pieces.append(z[:, 64 * r:64 * (r + 1)])
            pieces.append(mm(s, wint_ref[r]))
            rev_pieces.append(mm(s, wrev_ref[r]))
        fall = jnp.concatenate(pieces, axis=1) * c
        fid = mm(fall, fw[...]) + fb[...]
        fcom = mm(z[:, 320:640] * c, fcw[...]) + fcb[...]
        fdis = mm(z[:, 640:960] * c, fdw[...]) + fdb[...]
        frev = mm(jnp.concatenate(rev_pieces, axis=1) * c, frw[...]) + frb[...]
        return fid, fcom, fdis, frev

    zu = zu_ref[...]
    zi = zi_ref[...]
    cu = cu_ref[...]
    cii = cii_ref[...]
    fid_u, fc_u, fd_u, fr_u = side(zu, cu, fuw, fub, fucw, fucb, fudw, fudb,
                                   furw, furb)
    fid_i, fc_i, fd_i, fr_i = side(zi, cii, fiw, fib, ficw, ficb, fidw, fidb,
                                   firw, firb)

    def head(a, b, w1, w2):
        z = a * b
        return mm(jax.nn.relu(mm(z, w1[...])), w2[...])

    oi = head(fid_u, fid_i, piw1, piw2)
    orv = head(fr_u, fr_i, prw1, prw2)
    oc = head(fc_u, fc_i, pcw1, pcw2)
    od = head(fd_u, fd_i, pdw1, pdw2)
    sim = jnp.sum(mm(fc_u, pjw1[...]) * mm(fr_u, pjw2[...]), axis=1,
                  keepdims=True)
    out_ref[...] = jnp.concatenate([oi, orv, oc, od, sim], axis=1)


def _dense_block(zu, zi, cu, cii, wint, wrev, fcs, preds, proj):
    B = zu.shape[0]
    BLK = 512
    args = [zu, zi, cu, cii, wint, wrev] + fcs + preds + proj

    def whole(a):
        return pl.BlockSpec(a.shape, lambda i: (0,) * a.ndim)

    row_specs = [
        pl.BlockSpec((BLK, 1024), lambda i: (i, 0)),
        pl.BlockSpec((BLK, 1024), lambda i: (i, 0)),
        pl.BlockSpec((BLK, 1), lambda i: (i, 0)),
        pl.BlockSpec((BLK, 1), lambda i: (i, 0)),
    ]
    in_specs = row_specs + [whole(a) for a in args[4:]]
    return pl.pallas_call(
        _dense_body,
        grid=(B // BLK,),
        in_specs=in_specs,
        out_specs=pl.BlockSpec((BLK, 21), lambda i: (i, 0)),
        out_shape=jax.ShapeDtypeStruct((B, 21), jnp.float32),
    )(*args)


def kernel(edge_index, users, items, ci, review_feat, weight, weight_com,
           weight_dis, review_w_int, review_w_rev,
           fc_user_w, fc_user_b, fc_item_w, fc_item_b,
           fc_user_com_w, fc_user_com_b, fc_item_com_w, fc_item_com_b,
           fc_user_dis_w, fc_user_dis_b, fc_item_dis_w, fc_item_dis_b,
           fc_user_rev_w, fc_user_rev_b, fc_item_rev_w, fc_item_rev_b,
           pred_int_w1, pred_int_w2, pred_rev_w1, pred_rev_w2,
           pred_com_w1, pred_com_w2, pred_dis_w1, pred_dis_w2,
           proj_w1, proj_w2):
    src, dst = edge_index[0], edge_index[1]
    iid = N_USERS + items

    U = jnp.concatenate([weight, weight_com, weight_dis], axis=0)
    U = (U * ci[None, :, :]).transpose(1, 0, 2).reshape(N_NODES, 15 * EMB)

    g = ci[src]
    S = jax.ops.segment_sum(review_feat * g, dst, num_segments=N_NODES)
    Y = jax.ops.segment_sum(U[src], dst, num_segments=N_NODES)
    Z = jnp.concatenate([Y, S], axis=1)
    zu = Z[users]
    zi = Z[iid]
    cu = ci[users]
    cii = ci[iid]

    fcs = [fc_user_w, fc_user_b.reshape(1, -1), fc_item_w,
           fc_item_b.reshape(1, -1),
           fc_user_com_w, fc_user_com_b.reshape(1, -1), fc_item_com_w,
           fc_item_com_b.reshape(1, -1),
           fc_user_dis_w, fc_user_dis_b.reshape(1, -1), fc_item_dis_w,
           fc_item_dis_b.reshape(1, -1),
           fc_user_rev_w, fc_user_rev_b.reshape(1, -1), fc_item_rev_w,
           fc_item_rev_b.reshape(1, -1)]
    preds = [pred_int_w1, pred_int_w2, pred_rev_w1, pred_rev_w2,
             pred_com_w1, pred_com_w2, pred_dis_w1, pred_dis_w2]
    proj = [proj_w1, proj_w2]
    return _dense_block(zu, zi, cu, cii, review_w_int, review_w_rev, fcs,
                        preds, proj)


# speedup vs baseline: 6.5796x; 1.9920x over previous
import functools

import jax
import jax.numpy as jnp
from jax import lax
from jax.experimental import pallas as pl
from jax.experimental.pallas import tpu as pltpu
from jax.experimental.pallas import tpu_sc as plsc

N_USERS = 5000
N_ITEMS = 5000
N_NODES = N_USERS + N_ITEMS
EMB = 64
REV = 64
NW = 32
CHUNK = 80
GB = 128

_sc_mesh = functools.partial(
    plsc.VectorSubcoreMesh, core_axis_name="c", subcore_axis_name="s")


def _build_u_body(w_ref, wc_ref, wd_ref, ci_ref, u0, u1, u2, u3, u4, u5, u6,
                  u7, ci128):
    ci = ci_ref[...]
    blk = ci.shape[0]
    pieces = [w_ref[r] * ci for r in range(5)]
    pieces += [wc_ref[r] * ci for r in range(5)]
    pieces += [wd_ref[r] * ci for r in range(5)]
    u = jnp.concatenate(pieces, axis=1)
    outs = [u0, u1, u2, u3, u4, u5, u6]
    for b in range(7):
        outs[b][...] = u[:, 128 * b:128 * (b + 1)]
    u7[...] = jnp.concatenate(
        [u[:, 896:960], jnp.zeros((blk, 64), jnp.float32)], axis=1)
    ci128[...] = jnp.broadcast_to(ci, (blk, 128))


def _build_u(weight, weight_com, weight_dis, ci):
    n = ci.shape[0]
    blk = 2000
    w_spec = pl.BlockSpec((5, blk, EMB), lambda i: (0, i, 0))
    out_shapes = [jax.ShapeDtypeStruct((n, 128), jnp.float32)] * 9
    out_specs = [pl.BlockSpec((blk, 128), lambda i: (i, 0))] * 9
    return pl.pallas_call(
        _build_u_body,
        grid=(n // blk,),
        in_specs=[w_spec, w_spec, w_spec,
                  pl.BlockSpec((blk, 1), lambda i: (i, 0))],
        out_specs=out_specs,
        out_shape=out_shapes,
    )(weight, weight_com, weight_dis, ci)


def _gather_g(ci128, src):
    e = src.shape[0]
    e_per_w = e // NW
    n_ch = e_per_w // 40

    @functools.partial(
        pl.kernel,
        mesh=_sc_mesh(),
        out_type=jax.ShapeDtypeStruct((e, 128), jnp.float32),
        scratch_types=[pltpu.VMEM((40,), jnp.int32),
                       pltpu.VMEM((40, 128), jnp.float32),
                       pltpu.SemaphoreType.DMA],
    )
    def k(ci_hbm, src_hbm, g_hbm, idx_v, rows_v, sem):
        wid = lax.axis_index("s") * 2 + lax.axis_index("c")
        base = wid * e_per_w

        def body(ch, _):
            off = base + 40 * ch
            pltpu.sync_copy(src_hbm.at[pl.ds(off, 40)], idx_v)
            pltpu.async_copy(ci_hbm.at[idx_v], rows_v, sem).wait()
            pltpu.sync_copy(rows_v, g_hbm.at[pl.ds(off, 40)])
            return 0

        lax.fori_loop(0, n_ch, body, 0)

    return k(ci128, src)


def _build_rfg_body(rf_ref, g_ref, out_ref):
    blk = rf_ref.shape[0]
    out_ref[...] = jnp.concatenate(
        [rf_ref[...] * g_ref[:, :1], jnp.zeros((blk, 64), jnp.float32)],
        axis=1)


def _build_rfg(review_feat, g):
    e = review_feat.shape[0]
    blk = 8000
    return pl.pallas_call(
        _build_rfg_body,
        grid=(e // blk,),
        in_specs=[pl.BlockSpec((blk, REV), lambda i: (i, 0)),
                  pl.BlockSpec((blk, 128), lambda i: (i, 0))],
        out_specs=pl.BlockSpec((blk, 128), lambda i: (i, 0)),
        out_shape=jax.ShapeDtypeStruct((e, 128), jnp.float32),
    )(review_feat, g)


def _segment_sums(src, dst, us, rfg, zeros_a):
    e = src.shape[0]
    n = N_NODES
    e_per_t = e // 16
    n_ch = e_per_t // CHUNK
    n_pad = 10240
    rows_per_t = n_pad // 16

    out_type = [jax.ShapeDtypeStruct((n_pad, 128), jnp.float32)] * 9

    @functools.partial(
        pl.kernel,
        mesh=_sc_mesh(),
        out_type=out_type,
        scratch_types=[pltpu.VMEM((CHUNK,), jnp.int32),
                       pltpu.VMEM((CHUNK,), jnp.int32),
                       pltpu.VMEM((CHUNK, 128), jnp.float32),
                       pltpu.VMEM_SHARED((n_pad, 128), jnp.float32),
                       pltpu.SemaphoreType.DMA],
    )
    def k(src_hbm, dst_hbm, u0, u1, u2, u3, u4, u5, u6, u7, rfg_hbm,
          za_hbm,
          z0, z1, z2, z3, z4, z5, z6, z7, s_out,
          idxs, idxd, stage, acc, sem):
        core = lax.axis_index("c")
        tid = lax.axis_index("s")
        row0 = tid * rows_per_t
        ebase = tid * e_per_t

        def run_pass(tbl_hbm, out_hbm, is_gather):
            pltpu.sync_copy(za_hbm, acc.at[pl.ds(row0, rows_per_t)])
            plsc.subcore_barrier()

            def body(ch, _):
                off = ebase + CHUNK * ch
                pltpu.sync_copy(dst_hbm.at[pl.ds(off, CHUNK)], idxd)
                if is_gather:
                    pltpu.sync_copy(src_hbm.at[pl.ds(off, CHUNK)], idxs)
                    pltpu.async_copy(tbl_hbm.at[idxs], stage, sem).wait()
                else:
                    pltpu.sync_copy(tbl_hbm.at[pl.ds(off, CHUNK)], stage)
                pltpu.sync_copy(stage, acc.at[idxd], add=True)
                return 0

            lax.fori_loop(0, n_ch, body, 0)
            plsc.subcore_barrier()
            pltpu.sync_copy(acc.at[pl.ds(row0, rows_per_t)],
                            out_hbm.at[pl.ds(row0, rows_per_t)])
            plsc.subcore_barrier()

        @pl.when(core == 0)
        def _():
            run_pass(u0, z0, True)
            run_pass(u1, z1, True)
            run_pass(u2, z2, True)
            run_pass(u3, z3, True)

        @pl.when(core == 1)
        def _():
            run_pass(u4, z4, True)
            run_pass(u5, z5, True)
            run_pass(u6, z6, True)
            run_pass(u7, z7, True)
            run_pass(rfg_hbm, s_out, False)

    return k(src, dst, *us, rfg, zeros_a)


def _row_gathers(tables, users, iid):
    b = users.shape[0]
    nt = len(tables)
    out_type = [jax.ShapeDtypeStruct((b, 128), jnp.float32)] * (2 * nt)

    @functools.partial(
        pl.kernel,
        mesh=_sc_mesh(),
        out_type=out_type,
        scratch_types=[pltpu.VMEM((GB,), jnp.int32),
                       pltpu.VMEM((GB,), jnp.int32),
                       pltpu.VMEM((GB, 128), jnp.float32),
                       pltpu.SemaphoreType.DMA],
    )
    def k(*refs):
        tbls = refs[:nt]
        users_hbm, iid_hbm = refs[nt], refs[nt + 1]
        outs = refs[nt + 2:nt + 2 + 2 * nt]
        idx_u = refs[nt + 2 + 2 * nt]
        idx_i = refs[nt + 3 + 2 * nt]
        stg = refs[nt + 4 + 2 * nt]
        sem = refs[-1]
        wid = lax.axis_index("s") * 2 + lax.axis_index("c")
        off = wid * GB
        pltpu.sync_copy(users_hbm.at[pl.ds(off, GB)], idx_u)
        pltpu.sync_copy(iid_hbm.at[pl.ds(off, GB)], idx_i)
        for j in range(nt):
            pltpu.async_copy(tbls[j].at[idx_u], stg, sem).wait()
            pltpu.sync_copy(stg, outs[2 * j].at[pl.ds(off, GB)])
            pltpu.async_copy(tbls[j].at[idx_i], stg, sem).wait()
            pltpu.sync_copy(stg, outs[2 * j + 1].at[pl.ds(off, GB)])

    return k(*tables, users, iid)


def _dense_body(*refs):
    (zu0, zu1, zu2, zu3, zu4, zu5, zu6, zu7, su, cu,
     zi0, zi1, zi2, zi3, zi4, zi5, zi6, zi7, si, cii,
     wint_ref, wrev_ref,
     fuw, fub, fiw, fib, fucw, fucb, ficw, ficb,
     fudw, fudb, fidw, fidb, furw, furb, firw, firb,
     piw1, piw2, prw1, prw2, pcw1, pcw2, pdw1, pdw2,
     pjw1, pjw2, out_ref) = refs
    f32 = jnp.float32

    def mm(a, b):
        return jnp.dot(a, b, preferred_element_type=f32)

    def side(zrefs, z7_ref, s_ref, c_ref, fw, fb, fcw, fcb, fdw, fdb, frw,
             frb):
        z = jnp.concatenate([r[...] for r in zrefs] + [z7_ref[:, :64]],
                            axis=1)
        s = s_ref[:, :64]
        c = c_ref[:, :1]
        pieces = []
        rev_pieces = []
        for r in range(5):
            pieces.append(z[:, 64 * r:64 * (r + 1)])
            pieces.append(mm(s, wint_ref[r]))
            rev_pieces.append(mm(s, wrev_ref[r]))
        fall = jnp.concatenate(pieces, axis=1) * c
        fid = mm(fall, fw[...]) + fb[...]
        fcom = mm(z[:, 320:640] * c, fcw[...]) + fcb[...]
        fdis = mm(z[:, 640:960] * c, fdw[...]) + fdb[...]
        frev = mm(jnp.concatenate(rev_pieces, axis=1) * c, frw[...]) + frb[...]
        return fid, fcom, fdis, frev

    fid_u, fc_u, fd_u, fr_u = side(
        (zu0, zu1, zu2, zu3, zu4, zu5, zu6), zu7, su, cu,
        fuw, fub, fucw, fucb, fudw, fudb, furw, furb)
    fid_i, fc_i, fd_i, fr_i = side(
        (zi0, zi1, zi2, zi3, zi4, zi5, zi6), zi7, si, cii,
        fiw, fib, ficw, ficb, fidw, fidb, firw, firb)

    def head(a, b, w1, w2):
        z = a * b
        return mm(jax.nn.relu(mm(z, w1[...])), w2[...])

    oi = head(fid_u, fid_i, piw1, piw2)
    orv = head(fr_u, fr_i, prw1, prw2)
    oc = head(fc_u, fc_i, pcw1, pcw2)
    od = head(fd_u, fd_i, pdw1, pdw2)
    sim = jnp.sum(mm(fc_u, pjw1[...]) * mm(fr_u, pjw2[...]), axis=1,
                  keepdims=True)
    out_ref[...] = jnp.concatenate([oi, orv, oc, od, sim], axis=1)


def _dense_block(u_parts, i_parts, wint, wrev, fcs, preds, proj):
    b = u_parts[0].shape[0]
    blk = 512
    args = list(u_parts) + list(i_parts) + [wint, wrev] + fcs + preds + proj

    def whole(a):
        return pl.BlockSpec(a.shape, lambda i: (0,) * a.ndim)

    def rows(a):
        return pl.BlockSpec((blk, a.shape[1]), lambda i: (i, 0))

    in_specs = ([rows(a) for a in u_parts] + [rows(a) for a in i_parts]
                + [whole(a) for a in args[20:]])
    return pl.pallas_call(
        _dense_body,
        grid=(b // blk,),
        in_specs=in_specs,
        out_specs=pl.BlockSpec((blk, 21), lambda i: (i, 0)),
        out_shape=jax.ShapeDtypeStruct((b, 21), jnp.float32),
    )(*args)


def kernel(edge_index, users, items, ci, review_feat, weight, weight_com,
           weight_dis, review_w_int, review_w_rev,
           fc_user_w, fc_user_b, fc_item_w, fc_item_b,
           fc_user_com_w, fc_user_com_b, fc_item_com_w, fc_item_com_b,
           fc_user_dis_w, fc_user_dis_b, fc_item_dis_w, fc_item_dis_b,
           fc_user_rev_w, fc_user_rev_b, fc_item_rev_w, fc_item_rev_b,
           pred_int_w1, pred_int_w2, pred_rev_w1, pred_rev_w2,
           pred_com_w1, pred_com_w2, pred_dis_w1, pred_dis_w2,
           proj_w1, proj_w2):
    src = edge_index[0]
    dst = edge_index[1]
    iid = items + N_USERS

    *us, ci128 = _build_u(weight, weight_com, weight_dis, ci)
    g = _gather_g(ci128, src)
    rfg = _build_rfg(review_feat, g)

    zeros_a = jnp.zeros((640, 128), jnp.float32)
    zs = _segment_sums(src, dst, us, rfg, zeros_a)

    gathered = _row_gathers(list(zs) + [ci128], users, iid)
    u_parts = [gathered[2 * j] for j in range(10)]
    i_parts = [gathered[2 * j + 1] for j in range(10)]

    fcs = [fc_user_w, fc_user_b.reshape(1, -1), fc_item_w,
           fc_item_b.reshape(1, -1),
           fc_user_com_w, fc_user_com_b.reshape(1, -1), fc_item_com_w,
           fc_item_com_b.reshape(1, -1),
           fc_user_dis_w, fc_user_dis_b.reshape(1, -1), fc_item_dis_w,
           fc_item_dis_b.reshape(1, -1),
           fc_user_rev_w, fc_user_rev_b.reshape(1, -1), fc_item_rev_w,
           fc_item_rev_b.reshape(1, -1)]
    preds = [pred_int_w1, pred_int_w2, pred_rev_w1, pred_rev_w2,
             pred_com_w1, pred_com_w2, pred_dis_w1, pred_dis_w2]
    proj = [proj_w1, proj_w2]
    return _dense_block(u_parts, i_parts, review_w_int, review_w_rev, fcs,
                        preds, proj)


# speedup vs baseline: 9.7987x; 1.4892x over previous
import functools

import jax
import jax.numpy as jnp
from jax import lax
from jax.experimental import pallas as pl
from jax.experimental.pallas import tpu as pltpu
from jax.experimental.pallas import tpu_sc as plsc

N_USERS = 5000
N_ITEMS = 5000
N_NODES = N_USERS + N_ITEMS
EMB = 64
REV = 64
NW = 32
CHUNK = 80
GB = 128

_sc_mesh = functools.partial(
    plsc.VectorSubcoreMesh, core_axis_name="c", subcore_axis_name="s")


def _build_u_body(w_ref, wc_ref, wd_ref, ci_ref, u0, u1, u2, u3, u4, u5, u6,
                  u7, ci128):
    ci = ci_ref[...]
    blk = ci.shape[0]
    pieces = [w_ref[r] * ci for r in range(5)]
    pieces += [wc_ref[r] * ci for r in range(5)]
    pieces += [wd_ref[r] * ci for r in range(5)]
    u = jnp.concatenate(pieces, axis=1)
    outs = [u0, u1, u2, u3, u4, u5, u6]
    for b in range(7):
        outs[b][...] = u[:, 128 * b:128 * (b + 1)]
    u7[...] = jnp.concatenate(
        [u[:, 896:960], jnp.zeros((blk, 64), jnp.float32)], axis=1)
    ci128[...] = jnp.broadcast_to(ci, (blk, 128))


def _build_u(weight, weight_com, weight_dis, ci):
    n = ci.shape[0]
    blk = 2000
    w_spec = pl.BlockSpec((5, blk, EMB), lambda i: (0, i, 0))
    out_shapes = [jax.ShapeDtypeStruct((n, 128), jnp.float32)] * 9
    out_specs = [pl.BlockSpec((blk, 128), lambda i: (i, 0))] * 9
    return pl.pallas_call(
        _build_u_body,
        grid=(n // blk,),
        in_specs=[w_spec, w_spec, w_spec,
                  pl.BlockSpec((blk, 1), lambda i: (i, 0))],
        out_specs=out_specs,
        out_shape=out_shapes,
    )(weight, weight_com, weight_dis, ci)


def _gather_g(ci128, src):
    e = src.shape[0]
    e_per_w = e // NW
    n_ch = e_per_w // 40

    @functools.partial(
        pl.kernel,
        mesh=_sc_mesh(),
        out_type=jax.ShapeDtypeStruct((e, 128), jnp.float32),
        scratch_types=[pltpu.VMEM((40,), jnp.int32),
                       pltpu.VMEM((40, 128), jnp.float32),
                       pltpu.SemaphoreType.DMA],
    )
    def k(ci_hbm, src_hbm, g_hbm, idx_v, rows_v, sem):
        wid = lax.axis_index("s") * 2 + lax.axis_index("c")
        base = wid * e_per_w

        def body(ch, _):
            off = base + 40 * ch
            pltpu.sync_copy(src_hbm.at[pl.ds(off, 40)], idx_v)
            pltpu.async_copy(ci_hbm.at[idx_v], rows_v, sem).wait()
            pltpu.sync_copy(rows_v, g_hbm.at[pl.ds(off, 40)])
            return 0

        lax.fori_loop(0, n_ch, body, 0)

    return k(ci128, src)


def _build_rfg_body(rf_ref, g_ref, out_ref):
    blk = rf_ref.shape[0]
    out_ref[...] = jnp.concatenate(
        [rf_ref[...] * g_ref[:, :1], jnp.zeros((blk, 64), jnp.float32)],
        axis=1)


def _build_rfg(review_feat, g):
    e = review_feat.shape[0]
    blk = 8000
    return pl.pallas_call(
        _build_rfg_body,
        grid=(e // blk,),
        in_specs=[pl.BlockSpec((blk, REV), lambda i: (i, 0)),
                  pl.BlockSpec((blk, 128), lambda i: (i, 0))],
        out_specs=pl.BlockSpec((blk, 128), lambda i: (i, 0)),
        out_shape=jax.ShapeDtypeStruct((e, 128), jnp.float32),
    )(review_feat, g)


def _segment_sums(src, dst, us, rfg, zeros_a):
    e = src.shape[0]
    e_per_t = e // 16
    n_ch = e_per_t // CHUNK
    n_pad = 10240
    rows_per_t = n_pad // 16
    rf_split = (n_ch // 2) + 1

    out_type = [jax.ShapeDtypeStruct((n_pad, 128), jnp.float32)] * 10

    @functools.partial(
        pl.kernel,
        mesh=_sc_mesh(),
        out_type=out_type,
        scratch_types=[pltpu.VMEM((CHUNK,), jnp.int32),
                       pltpu.VMEM((CHUNK,), jnp.int32),
                       pltpu.VMEM((CHUNK,), jnp.int32),
                       pltpu.VMEM((CHUNK,), jnp.int32),
                       pltpu.VMEM((CHUNK, 128), jnp.float32),
                       pltpu.VMEM((CHUNK, 128), jnp.float32),
                       pltpu.VMEM_SHARED((n_pad, 128), jnp.float32),
                       pltpu.SemaphoreType.DMA,
                       pltpu.SemaphoreType.DMA,
                       pltpu.SemaphoreType.DMA,
                       pltpu.SemaphoreType.DMA,
                       pltpu.SemaphoreType.DMA,
                       pltpu.SemaphoreType.DMA],
    )
    def k(src_hbm, dst_hbm, u0, u1, u2, u3, u4, u5, u6, u7, rfg_hbm,
          za_hbm,
          z0, z1, z2, z3, z4, z5, z6, z7, s_a, s_b,
          idxs0, idxd0, idxs1, idxd1, stage0, stage1, acc,
          semi0, semi1, semg0, semg1, sems0, sems1):
        core = lax.axis_index("c")
        tid = lax.axis_index("s")
        row0 = tid * rows_per_t
        ebase = tid * e_per_t
        idxs = (idxs0, idxs1)
        idxd = (idxd0, idxd1)
        stage = (stage0, stage1)
        semi = (semi0, semi1)
        semg = (semg0, semg1)
        sems = (sems0, sems1)

        def run_pass(tbl_hbm, out_hbm, is_gather, c0, c1):
            nch = c1 - c0
            pltpu.sync_copy(za_hbm, acc.at[pl.ds(row0, rows_per_t)])
            plsc.subcore_barrier()

            def prime(c, b):
                off = jnp.minimum(ebase + CHUNK * c, e - CHUNK)
                if is_gather:
                    pltpu.async_copy(src_hbm.at[pl.ds(off, CHUNK)],
                                     idxs[b], semi[b])
                pltpu.async_copy(dst_hbm.at[pl.ds(off, CHUNK)],
                                 idxd[b], semi[b])

            def wait_idx(b):
                if is_gather:
                    pltpu.make_async_copy(src_hbm.at[pl.ds(0, CHUNK)],
                                          idxs[b], semi[b]).wait()
                pltpu.make_async_copy(dst_hbm.at[pl.ds(0, CHUNK)],
                                      idxd[b], semi[b]).wait()

            def start_fetch(c, b):
                if is_gather:
                    pltpu.async_copy(tbl_hbm.at[idxs[b]], stage[b], semg[b])
                else:
                    off = ebase + CHUNK * c
                    pltpu.async_copy(tbl_hbm.at[pl.ds(off, CHUNK)],
                                     stage[b], semg[b])

            def wait_fetch(b):
                pltpu.make_async_copy(rfg_hbm.at[pl.ds(0, CHUNK)],
                                      stage[b], semg[b]).wait()

            def start_scatter(b):
                pltpu.async_copy(stage[b], acc.at[idxd[b]], sems[b],
                                 add=True)

            def wait_scatter(b):
                pltpu.make_async_copy(stage[b], acc.at[idxd[b]],
                                      sems[b]).wait()

            prime(c0, 0)
            prime(c0 + 1, 1)

            def body(i, _):
                ca = c0 + 2 * i
                wait_idx(0)
                start_fetch(ca, 0)
                wait_idx(1)
                start_fetch(ca + 1, 1)
                wait_fetch(0)
                start_scatter(0)
                wait_fetch(1)
                start_scatter(1)
                wait_scatter(0)
                prime(ca + 2, 0)
                wait_scatter(1)
                prime(ca + 3, 1)
                return 0

            lax.fori_loop(0, nch // 2, body, 0)
            if nch % 2:
                wait_idx(0)
                start_fetch(c1 - 1, 0)
                wait_idx(1)
                wait_fetch(0)
                start_scatter(0)
                wait_scatter(0)
            else:
                wait_idx(0)
                wait_idx(1)
            plsc.subcore_barrier()
            pltpu.sync_copy(acc.at[pl.ds(row0, rows_per_t)],
                            out_hbm.at[pl.ds(row0, rows_per_t)])
            plsc.subcore_barrier()

        @pl.when(core == 0)
        def _():
            run_pass(u0, z0, True, 0, n_ch)
            run_pass(u1, z1, True, 0, n_ch)
            run_pass(u2, z2, True, 0, n_ch)
            run_pass(u3, z3, True, 0, n_ch)
            run_pass(rfg_hbm, s_a, False, 0, rf_split)

        @pl.when(core == 1)
        def _():
            run_pass(u4, z4, True, 0, n_ch)
            run_pass(u5, z5, True, 0, n_ch)
            run_pass(u6, z6, True, 0, n_ch)
            run_pass(u7, z7, True, 0, n_ch)
            run_pass(rfg_hbm, s_b, False, rf_split, n_ch)

    return k(src, dst, *us, rfg, zeros_a)


def _row_gathers(tables, users, iid):
    b = users.shape[0]
    nt = len(tables)
    out_type = [jax.ShapeDtypeStruct((b, 128), jnp.float32)] * (2 * nt)

    @functools.partial(
        pl.kernel,
        mesh=_sc_mesh(),
        out_type=out_type,
        scratch_types=[pltpu.VMEM((GB,), jnp.int32),
                       pltpu.VMEM((GB,), jnp.int32),
                       pltpu.VMEM((GB, 128), jnp.float32),
                       pltpu.SemaphoreType.DMA],
    )
    def k(*refs):
        tbls = refs[:nt]
        users_hbm, iid_hbm = refs[nt], refs[nt + 1]
        outs = refs[nt + 2:nt + 2 + 2 * nt]
        idx_u = refs[nt + 2 + 2 * nt]
        idx_i = refs[nt + 3 + 2 * nt]
        stg = refs[nt + 4 + 2 * nt]
        sem = refs[-1]
        wid = lax.axis_index("s") * 2 + lax.axis_index("c")
        off = wid * GB
        pltpu.sync_copy(users_hbm.at[pl.ds(off, GB)], idx_u)
        pltpu.sync_copy(iid_hbm.at[pl.ds(off, GB)], idx_i)
        for j in range(nt):
            pltpu.async_copy(tbls[j].at[idx_u], stg, sem).wait()
            pltpu.sync_copy(stg, outs[2 * j].at[pl.ds(off, GB)])
            pltpu.async_copy(tbls[j].at[idx_i], stg, sem).wait()
            pltpu.sync_copy(stg, outs[2 * j + 1].at[pl.ds(off, GB)])

    return k(*tables, users, iid)


def _dense_body(*refs):
    (zu0, zu1, zu2, zu3, zu4, zu5, zu6, zu7, sua, sub, cu,
     zi0, zi1, zi2, zi3, zi4, zi5, zi6, zi7, sia, sib, cii,
     wint_ref, wrev_ref,
     fuw, fub, fiw, fib, fucw, fucb, ficw, ficb,
     fudw, fudb, fidw, fidb, furw, furb, firw, firb,
     piw1, piw2, prw1, prw2, pcw1, pcw2, pdw1, pdw2,
     pjw1, pjw2, out_ref) = refs
    f32 = jnp.float32

    def mm(a, b):
        return jnp.dot(a, b, preferred_element_type=f32)

    def side(zrefs, z7_ref, sa_ref, sb_ref, c_ref, fw, fb, fcw, fcb, fdw,
             fdb, frw, frb):
        z = jnp.concatenate([r[...] for r in zrefs] + [z7_ref[:, :64]],
                            axis=1)
        s = sa_ref[:, :64] + sb_ref[:, :64]
        c = c_ref[:, :1]
        pieces = []
        rev_pieces = []
        for r in range(5):
            pieces.append(z[:, 64 * r:64 * (r + 1)])
            pieces.append(mm(s, wint_ref[r]))
            rev_pieces.append(mm(s, wrev_ref[r]))
        fall = jnp.concatenate(pieces, axis=1) * c
        fid = mm(fall, fw[...]) + fb[...]
        fcom = mm(z[:, 320:640] * c, fcw[...]) + fcb[...]
        fdis = mm(z[:, 640:960] * c, fdw[...]) + fdb[...]
        frev = mm(jnp.concatenate(rev_pieces, axis=1) * c, frw[...]) + frb[...]
        return fid, fcom, fdis, frev

    fid_u, fc_u, fd_u, fr_u = side(
        (zu0, zu1, zu2, zu3, zu4, zu5, zu6), zu7, sua, sub, cu,
        fuw, fub, fucw, fucb, fudw, fudb, furw, furb)
    fid_i, fc_i, fd_i, fr_i = side(
        (zi0, zi1, zi2, zi3, zi4, zi5, zi6), zi7, sia, sib, cii,
        fiw, fib, ficw, ficb, fidw, fidb, firw, firb)

    def head(a, b, w1, w2):
        z = a * b
        return mm(jax.nn.relu(mm(z, w1[...])), w2[...])

    oi = head(fid_u, fid_i, piw1, piw2)
    orv = head(fr_u, fr_i, prw1, prw2)
    oc = head(fc_u, fc_i, pcw1, pcw2)
    od = head(fd_u, fd_i, pdw1, pdw2)
    sim = jnp.sum(mm(fc_u, pjw1[...]) * mm(fr_u, pjw2[...]), axis=1,
                  keepdims=True)
    out_ref[...] = jnp.concatenate([oi, orv, oc, od, sim], axis=1)


def _dense_block(u_parts, i_parts, wint, wrev, fcs, preds, proj):
    b = u_parts[0].shape[0]
    blk = 512
    args = list(u_parts) + list(i_parts) + [wint, wrev] + fcs + preds + proj

    def whole(a):
        return pl.BlockSpec(a.shape, lambda i: (0,) * a.ndim)

    def rows(a):
        return pl.BlockSpec((blk, a.shape[1]), lambda i: (i, 0))

    in_specs = ([rows(a) for a in u_parts] + [rows(a) for a in i_parts]
                + [whole(a) for a in args[22:]])
    return pl.pallas_call(
        _dense_body,
        grid=(b // blk,),
        in_specs=in_specs,
        out_specs=pl.BlockSpec((blk, 21), lambda i: (i, 0)),
        out_shape=jax.ShapeDtypeStruct((b, 21), jnp.float32),
    )(*args)


def kernel(edge_index, users, items, ci, review_feat, weight, weight_com,
           weight_dis, review_w_int, review_w_rev,
           fc_user_w, fc_user_b, fc_item_w, fc_item_b,
           fc_user_com_w, fc_user_com_b, fc_item_com_w, fc_item_com_b,
           fc_user_dis_w, fc_user_dis_b, fc_item_dis_w, fc_item_dis_b,
           fc_user_rev_w, fc_user_rev_b, fc_item_rev_w, fc_item_rev_b,
           pred_int_w1, pred_int_w2, pred_rev_w1, pred_rev_w2,
           pred_com_w1, pred_com_w2, pred_dis_w1, pred_dis_w2,
           proj_w1, proj_w2):
    src = edge_index[0]
    dst = edge_index[1]
    iid = items + N_USERS

    *us, ci128 = _build_u(weight, weight_com, weight_dis, ci)
    g = _gather_g(ci128, src)
    rfg = _build_rfg(review_feat, g)

    zeros_a = jnp.zeros((640, 128), jnp.float32)
    zs = _segment_sums(src, dst, us, rfg, zeros_a)

    gathered = _row_gathers(list(zs) + [ci128], users, iid)
    u_parts = [gathered[2 * j] for j in range(11)]
    i_parts = [gathered[2 * j + 1] for j in range(11)]

    fcs = [fc_user_w, fc_user_b.reshape(1, -1), fc_item_w,
           fc_item_b.reshape(1, -1),
           fc_user_com_w, fc_user_com_b.reshape(1, -1), fc_item_com_w,
           fc_item_com_b.reshape(1, -1),
           fc_user_dis_w, fc_user_dis_b.reshape(1, -1), fc_item_dis_w,
           fc_item_dis_b.reshape(1, -1),
           fc_user_rev_w, fc_user_rev_b.reshape(1, -1), fc_item_rev_w,
           fc_item_rev_b.reshape(1, -1)]
    preds = [pred_int_w1, pred_int_w2, pred_rev_w1, pred_rev_w2,
             pred_com_w1, pred_com_w2, pred_dis_w1, pred_dis_w2]
    proj = [proj_w1, proj_w2]
    return _dense_block(u_parts, i_parts, review_w_int, review_w_rev, fcs,
                        preds, proj)


# speedup vs baseline: 10.8521x; 1.1075x over previous
import functools

import jax
import jax.numpy as jnp
from jax import lax
from jax.experimental import pallas as pl
from jax.experimental.pallas import tpu as pltpu
from jax.experimental.pallas import tpu_sc as plsc

N_USERS = 5000
N_ITEMS = 5000
N_NODES = N_USERS + N_ITEMS
EMB = 64
REV = 64
NW = 32
CHUNK = 80
GB = 128

_sc_mesh = functools.partial(
    plsc.VectorSubcoreMesh, core_axis_name="c", subcore_axis_name="s")


def _build_u_body(w_ref, wc_ref, wd_ref, ci_ref, u0, u1, u2, u3, u4, u5, u6,
                  u7):
    ci = ci_ref[...]
    blk = ci.shape[0]
    pieces = [w_ref[r] * ci for r in range(5)]
    pieces += [wc_ref[r] * ci for r in range(5)]
    pieces += [wd_ref[r] * ci for r in range(5)]
    u = jnp.concatenate(pieces, axis=1)
    outs = [u0, u1, u2, u3, u4, u5, u6]
    for b in range(7):
        outs[b][...] = u[:, 128 * b:128 * (b + 1)]
    u7[...] = jnp.concatenate(
        [u[:, 896:960], jnp.zeros((blk, 64), jnp.float32)], axis=1)


def _build_u(weight, weight_com, weight_dis, ci):
    n = ci.shape[0]
    blk = 2000
    w_spec = pl.BlockSpec((5, blk, EMB), lambda i: (0, i, 0))
    out_shapes = [jax.ShapeDtypeStruct((n, 128), jnp.float32)] * 8
    out_specs = [pl.BlockSpec((blk, 128), lambda i: (i, 0))] * 8
    return pl.pallas_call(
        _build_u_body,
        grid=(n // blk,),
        in_specs=[w_spec, w_spec, w_spec,
                  pl.BlockSpec((blk, 1), lambda i: (i, 0))],
        out_specs=out_specs,
        out_shape=out_shapes,
    )(weight, weight_com, weight_dis, ci)


def _build_ci128_body(ci_ref, out_ref):
    out_ref[...] = jnp.broadcast_to(ci_ref[...], out_ref.shape)


def _build_ci128(ci):
    n = ci.shape[0]
    blk = 2000
    return pl.pallas_call(
        _build_ci128_body,
        grid=(n // blk,),
        in_specs=[pl.BlockSpec((blk, 1), lambda i: (i, 0))],
        out_specs=pl.BlockSpec((blk, 128), lambda i: (i, 0)),
        out_shape=jax.ShapeDtypeStruct((n, 128), jnp.float32),
    )(ci)


def _build_rfg_sc(ci128, review_feat, src):
    e = src.shape[0]
    e_per_w = e // NW
    C = 40
    n_ch = e_per_w // C

    @functools.partial(
        pl.kernel,
        mesh=_sc_mesh(),
        out_type=jax.ShapeDtypeStruct((e, 128), jnp.float32),
        scratch_types=[pltpu.VMEM((C,), jnp.int32),
                       pltpu.VMEM((C,), jnp.int32),
                       pltpu.VMEM((C, 128), jnp.float32),
                       pltpu.VMEM((C, 128), jnp.float32),
                       pltpu.VMEM((C, REV), jnp.float32),
                       pltpu.VMEM((C, REV), jnp.float32),
                       pltpu.VMEM((C, 128), jnp.float32),
                       pltpu.VMEM((C, 128), jnp.float32),
                       pltpu.SemaphoreType.DMA,
                       pltpu.SemaphoreType.DMA,
                       pltpu.SemaphoreType.DMA,
                       pltpu.SemaphoreType.DMA,
                       pltpu.SemaphoreType.DMA,
                       pltpu.SemaphoreType.DMA,
                       pltpu.SemaphoreType.DMA,
                       pltpu.SemaphoreType.DMA],
    )
    def k(ci_hbm, rf_hbm, src_hbm, rfg_hbm,
          idx0, idx1, civ0, civ1, rfv0, rfv1, outv0, outv1,
          semi0, semi1, semg0, semg1, semr0, semr1, semw0, semw1):
        wid = lax.axis_index("s") * 2 + lax.axis_index("c")
        base = wid * e_per_w
        idx = (idx0, idx1)
        civ = (civ0, civ1)
        rfv = (rfv0, rfv1)
        outv = (outv0, outv1)
        semi = (semi0, semi1)
        semg = (semg0, semg1)
        semr = (semr0, semr1)
        semw = (semw0, semw1)

        zero16 = jnp.zeros((16,), jnp.float32)
        for b in range(2):
            for j in range(C):
                for q in range(4):
                    outv[b][j, pl.ds(64 + 16 * q, 16)] = zero16

        def prime(c, b):
            off = jnp.minimum(base + C * c, e - C)
            pltpu.async_copy(src_hbm.at[pl.ds(off, C)], idx[b], semi[b])

        def start_fetch(c, b):
            off = base + C * c
            pltpu.make_async_copy(src_hbm.at[pl.ds(0, C)], idx[b],
                                  semi[b]).wait()
            pltpu.async_copy(ci_hbm.at[idx[b]], civ[b], semg[b])
            pltpu.async_copy(rf_hbm.at[pl.ds(off, C)], rfv[b], semr[b])

        def compute_write(c, b):
            off = base + C * c
            pltpu.make_async_copy(ci_hbm.at[idx[b]], civ[b], semg[b]).wait()
            pltpu.make_async_copy(rf_hbm.at[pl.ds(0, C)], rfv[b],
                                  semr[b]).wait()
            for j in range(C):
                for q in range(4):
                    sl = pl.ds(16 * q, 16)
                    outv[b][j, sl] = rfv[b][j, sl] * civ[b][j, sl]
            pltpu.async_copy(outv[b], rfg_hbm.at[pl.ds(off, C)], semw[b])

        def reuse(c, b):
            pltpu.make_async_copy(outv[b], rfg_hbm.at[pl.ds(0, C)],
                                  semw[b]).wait()
            prime(c, b)

        prime(0, 0)
        prime(1, 1)

        def body(i, _):
            ca = 2 * i
            start_fetch(ca, 0)
            start_fetch(ca + 1, 1)
            compute_write(ca, 0)
            compute_write(ca + 1, 1)
            reuse(ca + 2, 0)
            reuse(ca + 3, 1)
            return 0

        lax.fori_loop(0, n_ch // 2, body, 0)
        if n_ch % 2:
            start_fetch(n_ch - 1, 0)
            pltpu.make_async_copy(src_hbm.at[pl.ds(0, C)], idx[1],
                                  semi[1]).wait()
            compute_write(n_ch - 1, 0)
            pltpu.make_async_copy(outv[0], rfg_hbm.at[pl.ds(0, C)],
                                  semw[0]).wait()
        else:
            pltpu.make_async_copy(src_hbm.at[pl.ds(0, C)], idx[0],
                                  semi[0]).wait()
            pltpu.make_async_copy(src_hbm.at[pl.ds(0, C)], idx[1],
                                  semi[1]).wait()

    return k(ci128, review_feat, src)


def _segment_sums(src, dst, us, rfg, zeros_a):
    e = src.shape[0]
    e_per_t = e // 16
    n_ch = e_per_t // CHUNK
    n_pad = 10240
    rows_per_t = n_pad // 16
    rf_split = (n_ch // 2) + 1

    out_type = [jax.ShapeDtypeStruct((n_pad, 128), jnp.float32)] * 10

    @functools.partial(
        pl.kernel,
        mesh=_sc_mesh(),
        out_type=out_type,
        scratch_types=[pltpu.VMEM((CHUNK,), jnp.int32),
                       pltpu.VMEM((CHUNK,), jnp.int32),
                       pltpu.VMEM((CHUNK,), jnp.int32),
                       pltpu.VMEM((CHUNK,), jnp.int32),
                       pltpu.VMEM((CHUNK, 128), jnp.float32),
                       pltpu.VMEM((CHUNK, 128), jnp.float32),
                       pltpu.VMEM_SHARED((n_pad, 128), jnp.float32),
                       pltpu.SemaphoreType.DMA,
                       pltpu.SemaphoreType.DMA,
                       pltpu.SemaphoreType.DMA,
                       pltpu.SemaphoreType.DMA,
                       pltpu.SemaphoreType.DMA,
                       pltpu.SemaphoreType.DMA],
    )
    def k(src_hbm, dst_hbm, u0, u1, u2, u3, u4, u5, u6, u7, rfg_hbm,
          za_hbm,
          z0, z1, z2, z3, z4, z5, z6, z7, s_a, s_b,
          idxs0, idxd0, idxs1, idxd1, stage0, stage1, acc,
          semi0, semi1, semg0, semg1, sems0, sems1):
        core = lax.axis_index("c")
        tid = lax.axis_index("s")
        row0 = tid * rows_per_t
        ebase = tid * e_per_t
        idxs = (idxs0, idxs1)
        idxd = (idxd0, idxd1)
        stage = (stage0, stage1)
        semi = (semi0, semi1)
        semg = (semg0, semg1)
        sems = (sems0, sems1)

        def run_pass(tbl_hbm, out_hbm, is_gather, c0, c1):
            nch = c1 - c0
            pltpu.sync_copy(za_hbm, acc.at[pl.ds(row0, rows_per_t)])
            plsc.subcore_barrier()

            def prime(c, b):
                off = jnp.minimum(ebase + CHUNK * c, e - CHUNK)
                if is_gather:
                    pltpu.async_copy(src_hbm.at[pl.ds(off, CHUNK)],
                                     idxs[b], semi[b])
                pltpu.async_copy(dst_hbm.at[pl.ds(off, CHUNK)],
                                 idxd[b], semi[b])

            def wait_idx(b):
                if is_gather:
                    pltpu.make_async_copy(src_hbm.at[pl.ds(0, CHUNK)],
                                          idxs[b], semi[b]).wait()
                pltpu.make_async_copy(dst_hbm.at[pl.ds(0, CHUNK)],
                                      idxd[b], semi[b]).wait()

            def start_fetch(c, b):
                if is_gather:
                    pltpu.async_copy(tbl_hbm.at[idxs[b]], stage[b], semg[b])
                else:
                    off = ebase + CHUNK * c
                    pltpu.async_copy(tbl_hbm.at[pl.ds(off, CHUNK)],
                                     stage[b], semg[b])

            def wait_fetch(b):
                pltpu.make_async_copy(rfg_hbm.at[pl.ds(0, CHUNK)],
                                      stage[b], semg[b]).wait()

            def start_scatter(b):
                pltpu.async_copy(stage[b], acc.at[idxd[b]], sems[b],
                                 add=True)

            def wait_scatter(b):
                pltpu.make_async_copy(stage[b], acc.at[idxd[b]],
                                      sems[b]).wait()

            prime(c0, 0)
            prime(c0 + 1, 1)

            def body(i, _):
                ca = c0 + 2 * i
                wait_idx(0)
                start_fetch(ca, 0)
                wait_idx(1)
                start_fetch(ca + 1, 1)
                wait_fetch(0)
                start_scatter(0)
                wait_fetch(1)
                start_scatter(1)
                wait_scatter(0)
                prime(ca + 2, 0)
                wait_scatter(1)
                prime(ca + 3, 1)
                return 0

            lax.fori_loop(0, nch // 2, body, 0)
            if nch % 2:
                wait_idx(0)
                start_fetch(c1 - 1, 0)
                wait_idx(1)
                wait_fetch(0)
                start_scatter(0)
                wait_scatter(0)
            else:
                wait_idx(0)
                wait_idx(1)
            plsc.subcore_barrier()
            pltpu.sync_copy(acc.at[pl.ds(row0, rows_per_t)],
                            out_hbm.at[pl.ds(row0, rows_per_t)])
            plsc.subcore_barrier()

        @pl.when(core == 0)
        def _():
            run_pass(u0, z0, True, 0, n_ch)
            run_pass(u1, z1, True, 0, n_ch)
            run_pass(u2, z2, True, 0, n_ch)
            run_pass(u3, z3, True, 0, n_ch)
            run_pass(rfg_hbm, s_a, False, 0, rf_split)

        @pl.when(core == 1)
        def _():
            run_pass(u4, z4, True, 0, n_ch)
            run_pass(u5, z5, True, 0, n_ch)
            run_pass(u6, z6, True, 0, n_ch)
            run_pass(u7, z7, True, 0, n_ch)
            run_pass(rfg_hbm, s_b, False, rf_split, n_ch)

    return k(src, dst, *us, rfg, zeros_a)


def _row_gathers(tables, users, iid):
    b = users.shape[0]
    nt = len(tables)
    out_type = [jax.ShapeDtypeStruct((b, 128), jnp.float32)] * (2 * nt)

    @functools.partial(
        pl.kernel,
        mesh=_sc_mesh(),
        out_type=out_type,
        scratch_types=[pltpu.VMEM((GB,), jnp.int32),
                       pltpu.VMEM((GB,), jnp.int32),
                       pltpu.VMEM((GB, 128), jnp.float32),
                       pltpu.SemaphoreType.DMA],
    )
    def k(*refs):
        tbls = refs[:nt]
        users_hbm, iid_hbm = refs[nt], refs[nt + 1]
        outs = refs[nt + 2:nt + 2 + 2 * nt]
        idx_u = refs[nt + 2 + 2 * nt]
        idx_i = refs[nt + 3 + 2 * nt]
        stg = refs[nt + 4 + 2 * nt]
        sem = refs[-1]
        wid = lax.axis_index("s") * 2 + lax.axis_index("c")
        off = wid * GB
        pltpu.sync_copy(users_hbm.at[pl.ds(off, GB)], idx_u)
        pltpu.sync_copy(iid_hbm.at[pl.ds(off, GB)], idx_i)
        for j in range(nt):
            pltpu.async_copy(tbls[j].at[idx_u], stg, sem).wait()
            pltpu.sync_copy(stg, outs[2 * j].at[pl.ds(off, GB)])
            pltpu.async_copy(tbls[j].at[idx_i], stg, sem).wait()
            pltpu.sync_copy(stg, outs[2 * j + 1].at[pl.ds(off, GB)])

    return k(*tables, users, iid)


def _dense_body(*refs):
    (zu0, zu1, zu2, zu3, zu4, zu5, zu6, zu7, sua, sub, cu,
     zi0, zi1, zi2, zi3, zi4, zi5, zi6, zi7, sia, sib, cii,
     wint_ref, wrev_ref,
     fuw, fub, fiw, fib, fucw, fucb, ficw, ficb,
     fudw, fudb, fidw, fidb, furw, furb, firw, firb,
     piw1, piw2, prw1, prw2, pcw1, pcw2, pdw1, pdw2,
     pjw1, pjw2, out_ref) = refs
    f32 = jnp.float32

    def mm(a, b):
        return jnp.dot(a, b, preferred_element_type=f32)

    def side(zrefs, z7_ref, sa_ref, sb_ref, c_ref, fw, fb, fcw, fcb, fdw,
             fdb, frw, frb):
        z = jnp.concatenate([r[...] for r in zrefs] + [z7_ref[:, :64]],
                            axis=1)
        s = sa_ref[:, :64] + sb_ref[:, :64]
        c = c_ref[:, :1]
        pieces = []
        rev_pieces = []
        for r in range(5):
            pieces.append(z[:, 64 * r:64 * (r + 1)])
            pieces.append(mm(s, wint_ref[r]))
            rev_pieces.append(mm(s, wrev_ref[r]))
        fall = jnp.concatenate(pieces, axis=1) * c
        fid = mm(fall, fw[...]) + fb[...]
        fcom = mm(z[:, 320:640] * c, fcw[...]) + fcb[...]
        fdis = mm(z[:, 640:960] * c, fdw[...]) + fdb[...]
        frev = mm(jnp.concatenate(rev_pieces, axis=1) * c, frw[...]) + frb[...]
        return fid, fcom, fdis, frev

    fid_u, fc_u, fd_u, fr_u = side(
        (zu0, zu1, zu2, zu3, zu4, zu5, zu6), zu7, sua, sub, cu,
        fuw, fub, fucw, fucb, fudw, fudb, furw, furb)
    fid_i, fc_i, fd_i, fr_i = side(
        (zi0, zi1, zi2, zi3, zi4, zi5, zi6), zi7, sia, sib, cii,
        fiw, fib, ficw, ficb, fidw, fidb, firw, firb)

    def head(a, b, w1, w2):
        z = a * b
        return mm(jax.nn.relu(mm(z, w1[...])), w2[...])

    oi = head(fid_u, fid_i, piw1, piw2)
    orv = head(fr_u, fr_i, prw1, prw2)
    oc = head(fc_u, fc_i, pcw1, pcw2)
    od = head(fd_u, fd_i, pdw1, pdw2)
    sim = jnp.sum(mm(fc_u, pjw1[...]) * mm(fr_u, pjw2[...]), axis=1,
                  keepdims=True)
    out_ref[...] = jnp.concatenate([oi, orv, oc, od, sim], axis=1)


def _dense_block(u_parts, i_parts, wint, wrev, fcs, preds, proj):
    b = u_parts[0].shape[0]
    blk = 512
    args = list(u_parts) + list(i_parts) + [wint, wrev] + fcs + preds + proj

    def whole(a):
        return pl.BlockSpec(a.shape, lambda i: (0,) * a.ndim)

    def rows(a):
        return pl.BlockSpec((blk, a.shape[1]), lambda i: (i, 0))

    in_specs = ([rows(a) for a in u_parts] + [rows(a) for a in i_parts]
                + [whole(a) for a in args[22:]])
    return pl.pallas_call(
        _dense_body,
        grid=(b // blk,),
        in_specs=in_specs,
        out_specs=pl.BlockSpec((blk, 21), lambda i: (i, 0)),
        out_shape=jax.ShapeDtypeStruct((b, 21), jnp.float32),
    )(*args)


def kernel(edge_index, users, items, ci, review_feat, weight, weight_com,
           weight_dis, review_w_int, review_w_rev,
           fc_user_w, fc_user_b, fc_item_w, fc_item_b,
           fc_user_com_w, fc_user_com_b, fc_item_com_w, fc_item_com_b,
           fc_user_dis_w, fc_user_dis_b, fc_item_dis_w, fc_item_dis_b,
           fc_user_rev_w, fc_user_rev_b, fc_item_rev_w, fc_item_rev_b,
           pred_int_w1, pred_int_w2, pred_rev_w1, pred_rev_w2,
           pred_com_w1, pred_com_w2, pred_dis_w1, pred_dis_w2,
           proj_w1, proj_w2):
    src = edge_index[0]
    dst = edge_index[1]
    iid = items + N_USERS

    ci128 = _build_ci128(ci)
    rfg = _build_rfg_sc(ci128, review_feat, src)
    us = _build_u(weight, weight_com, weight_dis, ci)

    zeros_a = jnp.zeros((640, 128), jnp.float32)
    zs = _segment_sums(src, dst, us, rfg, zeros_a)

    gathered = _row_gathers(list(zs) + [ci128], users, iid)
    u_parts = [gathered[2 * j] for j in range(11)]
    i_parts = [gathered[2 * j + 1] for j in range(11)]

    fcs = [fc_user_w, fc_user_b.reshape(1, -1), fc_item_w,
           fc_item_b.reshape(1, -1),
           fc_user_com_w, fc_user_com_b.reshape(1, -1), fc_item_com_w,
           fc_item_com_b.reshape(1, -1),
           fc_user_dis_w, fc_user_dis_b.reshape(1, -1), fc_item_dis_w,
           fc_item_dis_b.reshape(1, -1),
           fc_user_rev_w, fc_user_rev_b.reshape(1, -1), fc_item_rev_w,
           fc_item_rev_b.reshape(1, -1)]
    preds = [pred_int_w1, pred_int_w2, pred_rev_w1, pred_rev_w2,
             pred_com_w1, pred_com_w2, pred_dis_w1, pred_dis_w2]
    proj = [proj_w1, proj_w2]
    return _dense_block(u_parts, i_parts, review_w_int, review_w_rev, fcs,
                        preds, proj)


# speedup vs baseline: 12.8476x; 1.1839x over previous
import functools

import jax
import jax.numpy as jnp
from jax import lax
from jax.experimental import pallas as pl
from jax.experimental.pallas import tpu as pltpu
from jax.experimental.pallas import tpu_sc as plsc

N_USERS = 5000
N_ITEMS = 5000
N_NODES = N_USERS + N_ITEMS
EMB = 64
REV = 64
NW = 32
CHUNK = 80
GB = 128

_sc_mesh = functools.partial(
    plsc.VectorSubcoreMesh, core_axis_name="c", subcore_axis_name="s")


def _build_u_body(w_ref, wc_ref, wd_ref, ci_ref, u0, u1, u2, u3, u4, u5, u6,
                  u7):
    ci = ci_ref[...]
    blk = ci.shape[0]
    pieces = [w_ref[r] * ci for r in range(5)]
    pieces += [wc_ref[r] * ci for r in range(5)]
    pieces += [wd_ref[r] * ci for r in range(5)]
    u = jnp.concatenate(pieces, axis=1)
    outs = [u0, u1, u2, u3, u4, u5, u6]
    for b in range(7):
        outs[b][...] = u[:, 128 * b:128 * (b + 1)]
    u7[...] = jnp.concatenate(
        [u[:, 896:960], jnp.zeros((blk, 64), jnp.float32)], axis=1)


def _build_u(weight, weight_com, weight_dis, ci):
    n = ci.shape[0]
    blk = 2000
    w_spec = pl.BlockSpec((5, blk, EMB), lambda i: (0, i, 0))
    out_shapes = [jax.ShapeDtypeStruct((n, 128), jnp.float32)] * 8
    out_specs = [pl.BlockSpec((blk, 128), lambda i: (i, 0))] * 8
    return pl.pallas_call(
        _build_u_body,
        grid=(n // blk,),
        in_specs=[w_spec, w_spec, w_spec,
                  pl.BlockSpec((blk, 1), lambda i: (i, 0))],
        out_specs=out_specs,
        out_shape=out_shapes,
    )(weight, weight_com, weight_dis, ci)


def _build_ci128_body(ci_ref, out_ref):
    out_ref[...] = jnp.broadcast_to(ci_ref[...], out_ref.shape)


def _build_ci128(ci):
    n = ci.shape[0]
    blk = 2000
    return pl.pallas_call(
        _build_ci128_body,
        grid=(n // blk,),
        in_specs=[pl.BlockSpec((blk, 1), lambda i: (i, 0))],
        out_specs=pl.BlockSpec((blk, 128), lambda i: (i, 0)),
        out_shape=jax.ShapeDtypeStruct((n, 128), jnp.float32),
    )(ci)


def _build_rfg_sc(ci128, review_feat, src):
    e = src.shape[0]
    e_per_w = e // NW
    C = 40
    n_ch = e_per_w // C

    @functools.partial(
        pl.kernel,
        mesh=_sc_mesh(),
        out_type=jax.ShapeDtypeStruct((e, 128), jnp.float32),
        scratch_types=[pltpu.VMEM((C,), jnp.int32),
                       pltpu.VMEM((C,), jnp.int32),
                       pltpu.VMEM((C, 128), jnp.float32),
                       pltpu.VMEM((C, 128), jnp.float32),
                       pltpu.VMEM((C, REV), jnp.float32),
                       pltpu.VMEM((C, REV), jnp.float32),
                       pltpu.VMEM((C, 128), jnp.float32),
                       pltpu.VMEM((C, 128), jnp.float32),
                       pltpu.SemaphoreType.DMA,
                       pltpu.SemaphoreType.DMA,
                       pltpu.SemaphoreType.DMA,
                       pltpu.SemaphoreType.DMA,
                       pltpu.SemaphoreType.DMA,
                       pltpu.SemaphoreType.DMA,
                       pltpu.SemaphoreType.DMA,
                       pltpu.SemaphoreType.DMA],
    )
    def k(ci_hbm, rf_hbm, src_hbm, rfg_hbm,
          idx0, idx1, civ0, civ1, rfv0, rfv1, outv0, outv1,
          semi0, semi1, semg0, semg1, semr0, semr1, semw0, semw1):
        wid = lax.axis_index("s") * 2 + lax.axis_index("c")
        base = wid * e_per_w
        idx = (idx0, idx1)
        civ = (civ0, civ1)
        rfv = (rfv0, rfv1)
        outv = (outv0, outv1)
        semi = (semi0, semi1)
        semg = (semg0, semg1)
        semr = (semr0, semr1)
        semw = (semw0, semw1)

        zero16 = jnp.zeros((16,), jnp.float32)
        for b in range(2):
            for j in range(C):
                for q in range(4):
                    outv[b][j, pl.ds(64 + 16 * q, 16)] = zero16

        def prime(c, b):
            off = jnp.minimum(base + C * c, e - C)
            pltpu.async_copy(src_hbm.at[pl.ds(off, C)], idx[b], semi[b])

        def start_fetch(c, b):
            off = base + C * c
            pltpu.make_async_copy(src_hbm.at[pl.ds(0, C)], idx[b],
                                  semi[b]).wait()
            pltpu.async_copy(ci_hbm.at[idx[b]], civ[b], semg[b])
            pltpu.async_copy(rf_hbm.at[pl.ds(off, C)], rfv[b], semr[b])

        def compute_write(c, b):
            off = base + C * c
            pltpu.make_async_copy(ci_hbm.at[idx[b]], civ[b], semg[b]).wait()
            pltpu.make_async_copy(rf_hbm.at[pl.ds(0, C)], rfv[b],
                                  semr[b]).wait()
            for j in range(C):
                for q in range(4):
                    sl = pl.ds(16 * q, 16)
                    outv[b][j, sl] = rfv[b][j, sl] * civ[b][j, sl]
            pltpu.async_copy(outv[b], rfg_hbm.at[pl.ds(off, C)], semw[b])

        def reuse(c, b):
            pltpu.make_async_copy(outv[b], rfg_hbm.at[pl.ds(0, C)],
                                  semw[b]).wait()
            prime(c, b)

        prime(0, 0)
        prime(1, 1)

        def body(i, _):
            ca = 2 * i
            start_fetch(ca, 0)
            start_fetch(ca + 1, 1)
            compute_write(ca, 0)
            compute_write(ca + 1, 1)
            reuse(ca + 2, 0)
            reuse(ca + 3, 1)
            return 0

        lax.fori_loop(0, n_ch // 2, body, 0)
        if n_ch % 2:
            start_fetch(n_ch - 1, 0)
            pltpu.make_async_copy(src_hbm.at[pl.ds(0, C)], idx[1],
                                  semi[1]).wait()
            compute_write(n_ch - 1, 0)
            pltpu.make_async_copy(outv[0], rfg_hbm.at[pl.ds(0, C)],
                                  semw[0]).wait()
        else:
            pltpu.make_async_copy(src_hbm.at[pl.ds(0, C)], idx[0],
                                  semi[0]).wait()
            pltpu.make_async_copy(src_hbm.at[pl.ds(0, C)], idx[1],
                                  semi[1]).wait()

    return k(ci128, review_feat, src)


def _segment_sums(src, dst, us, rfg, zeros_a):
    e = src.shape[0]
    e_per_t = e // 16
    n_ch = e_per_t // CHUNK
    n_pad = 10240
    rows_per_t = n_pad // 16
    rf_split = (n_ch // 2) + 1

    out_type = [jax.ShapeDtypeStruct((n_pad, 128), jnp.float32)] * 10

    NB = 4

    @functools.partial(
        pl.kernel,
        mesh=_sc_mesh(),
        out_type=out_type,
        scratch_types=(
            [pltpu.VMEM((CHUNK,), jnp.int32)] * (2 * NB)
            + [pltpu.VMEM((CHUNK, 128), jnp.float32)] * NB
            + [pltpu.VMEM_SHARED((n_pad, 128), jnp.float32)]
            + [pltpu.SemaphoreType.DMA] * (3 * NB)),
    )
    def k(src_hbm, dst_hbm, u0, u1, u2, u3, u4, u5, u6, u7, rfg_hbm,
          za_hbm,
          z0, z1, z2, z3, z4, z5, z6, z7, s_a, s_b,
          *scr):
        idxs = scr[0:NB]
        idxd = scr[NB:2 * NB]
        stage = scr[2 * NB:3 * NB]
        acc = scr[3 * NB]
        semi = scr[3 * NB + 1:3 * NB + 1 + NB]
        semg = scr[3 * NB + 1 + NB:3 * NB + 1 + 2 * NB]
        sems = scr[3 * NB + 1 + 2 * NB:3 * NB + 1 + 3 * NB]
        core = lax.axis_index("c")
        tid = lax.axis_index("s")
        row0 = tid * rows_per_t
        ebase = tid * e_per_t

        def run_pass(tbl_hbm, out_hbm, is_gather, c0, c1):
            nch = c1 - c0
            pltpu.sync_copy(za_hbm, acc.at[pl.ds(row0, rows_per_t)])
            plsc.subcore_barrier()

            def prime(c, b):
                off = jnp.minimum(ebase + CHUNK * c, e - CHUNK)
                if is_gather:
                    pltpu.async_copy(src_hbm.at[pl.ds(off, CHUNK)],
                                     idxs[b], semi[b])
                pltpu.async_copy(dst_hbm.at[pl.ds(off, CHUNK)],
                                 idxd[b], semi[b])

            def wait_idx(b):
                if is_gather:
                    pltpu.make_async_copy(src_hbm.at[pl.ds(0, CHUNK)],
                                          idxs[b], semi[b]).wait()
                pltpu.make_async_copy(dst_hbm.at[pl.ds(0, CHUNK)],
                                      idxd[b], semi[b]).wait()

            def start_fetch(c, b):
                if is_gather:
                    pltpu.async_copy(tbl_hbm.at[idxs[b]], stage[b], semg[b])
                else:
                    off = ebase + CHUNK * c
                    pltpu.async_copy(tbl_hbm.at[pl.ds(off, CHUNK)],
                                     stage[b], semg[b])

            def wait_fetch(b):
                pltpu.make_async_copy(rfg_hbm.at[pl.ds(0, CHUNK)],
                                      stage[b], semg[b]).wait()

            def start_scatter(b):
                pltpu.async_copy(stage[b], acc.at[idxd[b]], sems[b],
                                 add=True)

            def wait_scatter(b):
                pltpu.make_async_copy(stage[b], acc.at[idxd[b]],
                                      sems[b]).wait()

            for b in range(NB):
                prime(c0 + b, b)

            def body(i, _):
                ca = c0 + NB * i
                for b in range(NB):
                    wait_idx(b)
                    start_fetch(ca + b, b)
                for b in range(NB):
                    wait_fetch(b)
                    start_scatter(b)
                for b in range(NB):
                    wait_scatter(b)
                    prime(ca + NB + b, b)
                return 0

            lax.fori_loop(0, nch // NB, body, 0)
            tail = nch % NB
            for b in range(tail):
                wait_idx(b)
                start_fetch(c1 - tail + b, b)
            for b in range(tail):
                wait_fetch(b)
                start_scatter(b)
            for b in range(tail):
                wait_scatter(b)
            for b in range(tail, NB):
                wait_idx(b)
            plsc.subcore_barrier()
            pltpu.sync_copy(acc.at[pl.ds(row0, rows_per_t)],
                            out_hbm.at[pl.ds(row0, rows_per_t)])
            plsc.subcore_barrier()

        @pl.when(core == 0)
        def _():
            run_pass(u0, z0, True, 0, n_ch)
            run_pass(u1, z1, True, 0, n_ch)
            run_pass(u2, z2, True, 0, n_ch)
            run_pass(u3, z3, True, 0, n_ch)
            run_pass(rfg_hbm, s_a, False, 0, rf_split)

        @pl.when(core == 1)
        def _():
            run_pass(u4, z4, True, 0, n_ch)
            run_pass(u5, z5, True, 0, n_ch)
            run_pass(u6, z6, True, 0, n_ch)
            run_pass(u7, z7, True, 0, n_ch)
            run_pass(rfg_hbm, s_b, False, rf_split, n_ch)

    return k(src, dst, *us, rfg, zeros_a)


def _row_gathers(tables, users, iid):
    b = users.shape[0]
    nt = len(tables)
    out_type = [jax.ShapeDtypeStruct((b, 128), jnp.float32)] * (2 * nt)

    @functools.partial(
        pl.kernel,
        mesh=_sc_mesh(),
        out_type=out_type,
        scratch_types=[pltpu.VMEM((GB,), jnp.int32),
                       pltpu.VMEM((GB,), jnp.int32),
                       pltpu.VMEM((GB, 128), jnp.float32),
                       pltpu.SemaphoreType.DMA],
    )
    def k(*refs):
        tbls = refs[:nt]
        users_hbm, iid_hbm = refs[nt], refs[nt + 1]
        outs = refs[nt + 2:nt + 2 + 2 * nt]
        idx_u = refs[nt + 2 + 2 * nt]
        idx_i = refs[nt + 3 + 2 * nt]
        stg = refs[nt + 4 + 2 * nt]
        sem = refs[-1]
        wid = lax.axis_index("s") * 2 + lax.axis_index("c")
        off = wid * GB
        pltpu.sync_copy(users_hbm.at[pl.ds(off, GB)], idx_u)
        pltpu.sync_copy(iid_hbm.at[pl.ds(off, GB)], idx_i)
        for j in range(nt):
            pltpu.async_copy(tbls[j].at[idx_u], stg, sem).wait()
            pltpu.sync_copy(stg, outs[2 * j].at[pl.ds(off, GB)])
            pltpu.async_copy(tbls[j].at[idx_i], stg, sem).wait()
            pltpu.sync_copy(stg, outs[2 * j + 1].at[pl.ds(off, GB)])

    return k(*tables, users, iid)


def _dense_body(*refs):
    (zu0, zu1, zu2, zu3, zu4, zu5, zu6, zu7, sua, sub, cu,
     zi0, zi1, zi2, zi3, zi4, zi5, zi6, zi7, sia, sib, cii,
     wint_ref, wrev_ref,
     fuw, fub, fiw, fib, fucw, fucb, ficw, ficb,
     fudw, fudb, fidw, fidb, furw, furb, firw, firb,
     piw1, piw2, prw1, prw2, pcw1, pcw2, pdw1, pdw2,
     pjw1, pjw2, out_ref) = refs
    f32 = jnp.float32

    def mm(a, b):
        return jnp.dot(a, b, preferred_element_type=f32)

    def side(zrefs, z7_ref, sa_ref, sb_ref, c_ref, fw, fb, fcw, fcb, fdw,
             fdb, frw, frb):
        z = jnp.concatenate([r[...] for r in zrefs] + [z7_ref[:, :64]],
                            axis=1)
        s = sa_ref[:, :64] + sb_ref[:, :64]
        c = c_ref[:, :1]
        pieces = []
        rev_pieces = []
        for r in range(5):
            pieces.append(z[:, 64 * r:64 * (r + 1)])
            pieces.append(mm(s, wint_ref[r]))
            rev_pieces.append(mm(s, wrev_ref[r]))
        fall = jnp.concatenate(pieces, axis=1) * c
        fid = mm(fall, fw[...]) + fb[...]
        fcom = mm(z[:, 320:640] * c, fcw[...]) + fcb[...]
        fdis = mm(z[:, 640:960] * c, fdw[...]) + fdb[...]
        frev = mm(jnp.concatenate(rev_pieces, axis=1) * c, frw[...]) + frb[...]
        return fid, fcom, fdis, frev

    fid_u, fc_u, fd_u, fr_u = side(
        (zu0, zu1, zu2, zu3, zu4, zu5, zu6), zu7, sua, sub, cu,
        fuw, fub, fucw, fucb, fudw, fudb, furw, furb)
    fid_i, fc_i, fd_i, fr_i = side(
        (zi0, zi1, zi2, zi3, zi4, zi5, zi6), zi7, sia, sib, cii,
        fiw, fib, ficw, ficb, fidw, fidb, firw, firb)

    def head(a, b, w1, w2):
        z = a * b
        return mm(jax.nn.relu(mm(z, w1[...])), w2[...])

    oi = head(fid_u, fid_i, piw1, piw2)
    orv = head(fr_u, fr_i, prw1, prw2)
    oc = head(fc_u, fc_i, pcw1, pcw2)
    od = head(fd_u, fd_i, pdw1, pdw2)
    sim = jnp.sum(mm(fc_u, pjw1[...]) * mm(fr_u, pjw2[...]), axis=1,
                  keepdims=True)
    out_ref[...] = jnp.concatenate([oi, orv, oc, od, sim], axis=1)


def _dense_block(u_parts, i_parts, wint, wrev, fcs, preds, proj):
    b = u_parts[0].shape[0]
    blk = 512
    args = list(u_parts) + list(i_parts) + [wint, wrev] + fcs + preds + proj

    def whole(a):
        return pl.BlockSpec(a.shape, lambda i: (0,) * a.ndim)

    def rows(a):
        return pl.BlockSpec((blk, a.shape[1]), lambda i: (i, 0))

    in_specs = ([rows(a) for a in u_parts] + [rows(a) for a in i_parts]
                + [whole(a) for a in args[22:]])
    return pl.pallas_call(
        _dense_body,
        grid=(b // blk,),
        in_specs=in_specs,
        out_specs=pl.BlockSpec((blk, 21), lambda i: (i, 0)),
        out_shape=jax.ShapeDtypeStruct((b, 21), jnp.float32),
    )(*args)


def kernel(edge_index, users, items, ci, review_feat, weight, weight_com,
           weight_dis, review_w_int, review_w_rev,
           fc_user_w, fc_user_b, fc_item_w, fc_item_b,
           fc_user_com_w, fc_user_com_b, fc_item_com_w, fc_item_com_b,
           fc_user_dis_w, fc_user_dis_b, fc_item_dis_w, fc_item_dis_b,
           fc_user_rev_w, fc_user_rev_b, fc_item_rev_w, fc_item_rev_b,
           pred_int_w1, pred_int_w2, pred_rev_w1, pred_rev_w2,
           pred_com_w1, pred_com_w2, pred_dis_w1, pred_dis_w2,
           proj_w1, proj_w2):
    src = edge_index[0]
    dst = edge_index[1]
    iid = items + N_USERS

    ci128 = _build_ci128(ci)
    rfg = _build_rfg_sc(ci128, review_feat, src)
    us = _build_u(weight, weight_com, weight_dis, ci)

    zeros_a = jnp.zeros((640, 128), jnp.float32)
    zs = _segment_sums(src, dst, us, rfg, zeros_a)

    gathered = _row_gathers(list(zs) + [ci128], users, iid)
    u_parts = [gathered[2 * j] for j in range(11)]
    i_parts = [gathered[2 * j + 1] for j in range(11)]

    fcs = [fc_user_w, fc_user_b.reshape(1, -1), fc_item_w,
           fc_item_b.reshape(1, -1),
           fc_user_com_w, fc_user_com_b.reshape(1, -1), fc_item_com_w,
           fc_item_com_b.reshape(1, -1),
           fc_user_dis_w, fc_user_dis_b.reshape(1, -1), fc_item_dis_w,
           fc_item_dis_b.reshape(1, -1),
           fc_user_rev_w, fc_user_rev_b.reshape(1, -1), fc_item_rev_w,
           fc_item_rev_b.reshape(1, -1)]
    preds = [pred_int_w1, pred_int_w2, pred_rev_w1, pred_rev_w2,
             pred_com_w1, pred_com_w2, pred_dis_w1, pred_dis_w2]
    proj = [proj_w1, proj_w2]
    return _dense_block(u_parts, i_parts, review_w_int, review_w_rev, fcs,
                        preds, proj)


# speedup vs baseline: 12.9751x; 1.0099x over previous
import functools

import jax
import jax.numpy as jnp
from jax import lax
from jax.experimental import pallas as pl
from jax.experimental.pallas import tpu as pltpu
from jax.experimental.pallas import tpu_sc as plsc

N_USERS = 5000
N_ITEMS = 5000
N_NODES = N_USERS + N_ITEMS
EMB = 64
REV = 64
NW = 32
CHUNK = 80
GB = 128

_sc_mesh = functools.partial(
    plsc.VectorSubcoreMesh, core_axis_name="c", subcore_axis_name="s")


def _build_u_body(w_ref, wc_ref, wd_ref, ci_ref, u0, u1, u2, u3, u4, u5, u6,
                  u7):
    ci = ci_ref[...]
    blk = ci.shape[0]
    pieces = [w_ref[r] * ci for r in range(5)]
    pieces += [wc_ref[r] * ci for r in range(5)]
    pieces += [wd_ref[r] * ci for r in range(5)]
    u = jnp.concatenate(pieces, axis=1)
    outs = [u0, u1, u2, u3, u4, u5, u6]
    for b in range(7):
        outs[b][...] = u[:, 128 * b:128 * (b + 1)]
    u7[...] = jnp.concatenate(
        [u[:, 896:960], jnp.zeros((blk, 64), jnp.float32)], axis=1)


def _build_u(weight, weight_com, weight_dis, ci):
    n = ci.shape[0]
    blk = 2000
    w_spec = pl.BlockSpec((5, blk, EMB), lambda i: (0, i, 0))
    out_shapes = [jax.ShapeDtypeStruct((n, 128), jnp.float32)] * 8
    out_specs = [pl.BlockSpec((blk, 128), lambda i: (i, 0))] * 8
    return pl.pallas_call(
        _build_u_body,
        grid=(n // blk,),
        in_specs=[w_spec, w_spec, w_spec,
                  pl.BlockSpec((blk, 1), lambda i: (i, 0))],
        out_specs=out_specs,
        out_shape=out_shapes,
    )(weight, weight_com, weight_dis, ci)


def _build_ci128_body(ci_ref, out_ref):
    out_ref[...] = jnp.broadcast_to(ci_ref[...], out_ref.shape)


def _build_ci128(ci):
    n = ci.shape[0]
    blk = 2000
    return pl.pallas_call(
        _build_ci128_body,
        grid=(n // blk,),
        in_specs=[pl.BlockSpec((blk, 1), lambda i: (i, 0))],
        out_specs=pl.BlockSpec((blk, 128), lambda i: (i, 0)),
        out_shape=jax.ShapeDtypeStruct((n, 128), jnp.float32),
    )(ci)


def _build_rfg_sc(ci128, review_feat, src):
    e = src.shape[0]
    e_per_w = e // NW
    C = 40
    n_ch = e_per_w // C

    NB = 4

    @functools.partial(
        pl.kernel,
        mesh=_sc_mesh(),
        out_type=jax.ShapeDtypeStruct((e, 128), jnp.float32),
        scratch_types=(
            [pltpu.VMEM((C,), jnp.int32)] * NB
            + [pltpu.VMEM((C, 128), jnp.float32)] * NB
            + [pltpu.VMEM((C, REV), jnp.float32)] * NB
            + [pltpu.VMEM((C, 128), jnp.float32)] * NB
            + [pltpu.SemaphoreType.DMA] * (4 * NB)),
    )
    def k(ci_hbm, rf_hbm, src_hbm, rfg_hbm, *scr):
        idx = scr[0:NB]
        civ = scr[NB:2 * NB]
        rfv = scr[2 * NB:3 * NB]
        outv = scr[3 * NB:4 * NB]
        semi = scr[4 * NB:5 * NB]
        semg = scr[5 * NB:6 * NB]
        semr = scr[6 * NB:7 * NB]
        semw = scr[7 * NB:8 * NB]
        wid = lax.axis_index("s") * 2 + lax.axis_index("c")
        base = wid * e_per_w

        zero16 = jnp.zeros((16,), jnp.float32)
        for b in range(NB):
            for j in range(C):
                for q in range(4):
                    outv[b][j, pl.ds(64 + 16 * q, 16)] = zero16

        def prime(c, b):
            off = jnp.minimum(base + C * c, e - C)
            pltpu.async_copy(src_hbm.at[pl.ds(off, C)], idx[b], semi[b])

        def start_fetch(c, b):
            off = base + C * c
            pltpu.make_async_copy(src_hbm.at[pl.ds(0, C)], idx[b],
                                  semi[b]).wait()
            pltpu.async_copy(ci_hbm.at[idx[b]], civ[b], semg[b])
            pltpu.async_copy(rf_hbm.at[pl.ds(off, C)], rfv[b], semr[b])

        def compute_write(c, b):
            off = base + C * c
            pltpu.make_async_copy(ci_hbm.at[idx[b]], civ[b], semg[b]).wait()
            pltpu.make_async_copy(rf_hbm.at[pl.ds(0, C)], rfv[b],
                                  semr[b]).wait()
            for j in range(C):
                for q in range(4):
                    sl = pl.ds(16 * q, 16)
                    outv[b][j, sl] = rfv[b][j, sl] * civ[b][j, sl]
            pltpu.async_copy(outv[b], rfg_hbm.at[pl.ds(off, C)], semw[b])

        def reuse(c, b):
            pltpu.make_async_copy(outv[b], rfg_hbm.at[pl.ds(0, C)],
                                  semw[b]).wait()
            prime(c, b)

        for b in range(NB):
            prime(b, b)

        def body(i, _):
            ca = NB * i
            for b in range(NB):
                start_fetch(ca + b, b)
            for b in range(NB):
                compute_write(ca + b, b)
            for b in range(NB):
                reuse(ca + NB + b, b)
            return 0

        lax.fori_loop(0, n_ch // NB, body, 0)
        tail = n_ch % NB
        for b in range(tail):
            start_fetch(n_ch - tail + b, b)
        for b in range(tail):
            compute_write(n_ch - tail + b, b)
        for b in range(tail):
            pltpu.make_async_copy(outv[b], rfg_hbm.at[pl.ds(0, C)],
                                  semw[b]).wait()
        for b in range(tail, NB):
            pltpu.make_async_copy(src_hbm.at[pl.ds(0, C)], idx[b],
                                  semi[b]).wait()

    return k(ci128, review_feat, src)


def _segment_sums(src, dst, us, rfg, zeros_a):
    e = src.shape[0]
    e_per_t = e // 16
    n_ch = e_per_t // CHUNK
    n_pad = 10240
    rows_per_t = n_pad // 16
    rf_split = (n_ch // 2) + 1

    out_type = [jax.ShapeDtypeStruct((n_pad, 128), jnp.float32)] * 10

    NB = 4

    @functools.partial(
        pl.kernel,
        mesh=_sc_mesh(),
        out_type=out_type,
        scratch_types=(
            [pltpu.VMEM((CHUNK,), jnp.int32)] * (2 * NB)
            + [pltpu.VMEM((CHUNK, 128), jnp.float32)] * NB
            + [pltpu.VMEM_SHARED((n_pad, 128), jnp.float32)]
            + [pltpu.SemaphoreType.DMA] * (3 * NB)),
    )
    def k(src_hbm, dst_hbm, u0, u1, u2, u3, u4, u5, u6, u7, rfg_hbm,
          za_hbm,
          z0, z1, z2, z3, z4, z5, z6, z7, s_a, s_b,
          *scr):
        idxs = scr[0:NB]
        idxd = scr[NB:2 * NB]
        stage = scr[2 * NB:3 * NB]
        acc = scr[3 * NB]
        semi = scr[3 * NB + 1:3 * NB + 1 + NB]
        semg = scr[3 * NB + 1 + NB:3 * NB + 1 + 2 * NB]
        sems = scr[3 * NB + 1 + 2 * NB:3 * NB + 1 + 3 * NB]
        core = lax.axis_index("c")
        tid = lax.axis_index("s")
        row0 = tid * rows_per_t
        ebase = tid * e_per_t

        def run_pass(tbl_hbm, out_hbm, is_gather, c0, c1):
            nch = c1 - c0
            pltpu.sync_copy(za_hbm, acc.at[pl.ds(row0, rows_per_t)])
            plsc.subcore_barrier()

            def prime(c, b):
                off = jnp.minimum(ebase + CHUNK * c, e - CHUNK)
                if is_gather:
                    pltpu.async_copy(src_hbm.at[pl.ds(off, CHUNK)],
                                     idxs[b], semi[b])
                pltpu.async_copy(dst_hbm.at[pl.ds(off, CHUNK)],
                                 idxd[b], semi[b])

            def wait_idx(b):
                if is_gather:
                    pltpu.make_async_copy(src_hbm.at[pl.ds(0, CHUNK)],
                                          idxs[b], semi[b]).wait()
                pltpu.make_async_copy(dst_hbm.at[pl.ds(0, CHUNK)],
                                      idxd[b], semi[b]).wait()

            def start_fetch(c, b):
                if is_gather:
                    pltpu.async_copy(tbl_hbm.at[idxs[b]], stage[b], semg[b])
                else:
                    off = ebase + CHUNK * c
                    pltpu.async_copy(tbl_hbm.at[pl.ds(off, CHUNK)],
                                     stage[b], semg[b])

            def wait_fetch(b):
                pltpu.make_async_copy(rfg_hbm.at[pl.ds(0, CHUNK)],
                                      stage[b], semg[b]).wait()

            def start_scatter(b):
                pltpu.async_copy(stage[b], acc.at[idxd[b]], sems[b],
                                 add=True)

            def wait_scatter(b):
                pltpu.make_async_copy(stage[b], acc.at[idxd[b]],
                                      sems[b]).wait()

            for b in range(NB):
                prime(c0 + b, b)

            def body(i, _):
                ca = c0 + NB * i
                for b in range(NB):
                    wait_idx(b)
                    start_fetch(ca + b, b)
                for b in range(NB):
                    wait_fetch(b)
                    start_scatter(b)
                for b in range(NB):
                    wait_scatter(b)
                    prime(ca + NB + b, b)
                return 0

            lax.fori_loop(0, nch // NB, body, 0)
            tail = nch % NB
            for b in range(tail):
                wait_idx(b)
                start_fetch(c1 - tail + b, b)
            for b in range(tail):
                wait_fetch(b)
                start_scatter(b)
            for b in range(tail):
                wait_scatter(b)
            for b in range(tail, NB):
                wait_idx(b)
            plsc.subcore_barrier()
            pltpu.sync_copy(acc.at[pl.ds(row0, rows_per_t)],
                            out_hbm.at[pl.ds(row0, rows_per_t)])
            plsc.subcore_barrier()

        @pl.when(core == 0)
        def _():
            run_pass(u0, z0, True, 0, n_ch)
            run_pass(u1, z1, True, 0, n_ch)
            run_pass(u2, z2, True, 0, n_ch)
            run_pass(u3, z3, True, 0, n_ch)
            run_pass(rfg_hbm, s_a, False, 0, rf_split)

        @pl.when(core == 1)
        def _():
            run_pass(u4, z4, True, 0, n_ch)
            run_pass(u5, z5, True, 0, n_ch)
            run_pass(u6, z6, True, 0, n_ch)
            run_pass(u7, z7, True, 0, n_ch)
            run_pass(rfg_hbm, s_b, False, rf_split, n_ch)

    return k(src, dst, *us, rfg, zeros_a)


def _row_gathers(tables, users, iid):
    b = users.shape[0]
    nt = len(tables)
    out_type = [jax.ShapeDtypeStruct((b, 128), jnp.float32)] * (2 * nt)

    @functools.partial(
        pl.kernel,
        mesh=_sc_mesh(),
        out_type=out_type,
        scratch_types=[pltpu.VMEM((GB,), jnp.int32),
                       pltpu.VMEM((GB,), jnp.int32),
                       pltpu.VMEM((GB, 128), jnp.float32),
                       pltpu.SemaphoreType.DMA],
    )
    def k(*refs):
        tbls = refs[:nt]
        users_hbm, iid_hbm = refs[nt], refs[nt + 1]
        outs = refs[nt + 2:nt + 2 + 2 * nt]
        idx_u = refs[nt + 2 + 2 * nt]
        idx_i = refs[nt + 3 + 2 * nt]
        stg = refs[nt + 4 + 2 * nt]
        sem = refs[-1]
        wid = lax.axis_index("s") * 2 + lax.axis_index("c")
        off = wid * GB
        pltpu.sync_copy(users_hbm.at[pl.ds(off, GB)], idx_u)
        pltpu.sync_copy(iid_hbm.at[pl.ds(off, GB)], idx_i)
        for j in range(nt):
            pltpu.async_copy(tbls[j].at[idx_u], stg, sem).wait()
            pltpu.sync_copy(stg, outs[2 * j].at[pl.ds(off, GB)])
            pltpu.async_copy(tbls[j].at[idx_i], stg, sem).wait()
            pltpu.sync_copy(stg, outs[2 * j + 1].at[pl.ds(off, GB)])

    return k(*tables, users, iid)


def _dense_body(*refs):
    (zu0, zu1, zu2, zu3, zu4, zu5, zu6, zu7, sua, sub, cu,
     zi0, zi1, zi2, zi3, zi4, zi5, zi6, zi7, sia, sib, cii,
     wint_ref, wrev_ref,
     fuw, fub, fiw, fib, fucw, fucb, ficw, ficb,
     fudw, fudb, fidw, fidb, furw, furb, firw, firb,
     piw1, piw2, prw1, prw2, pcw1, pcw2, pdw1, pdw2,
     pjw1, pjw2, out_ref) = refs
    f32 = jnp.float32

    def mm(a, b):
        return jnp.dot(a, b, preferred_element_type=f32)

    def side(zrefs, z7_ref, sa_ref, sb_ref, c_ref, fw, fb, fcw, fcb, fdw,
             fdb, frw, frb):
        z = jnp.concatenate([r[...] for r in zrefs] + [z7_ref[:, :64]],
                            axis=1)
        s = sa_ref[:, :64] + sb_ref[:, :64]
        c = c_ref[:, :1]
        pieces = []
        rev_pieces = []
        for r in range(5):
            pieces.append(z[:, 64 * r:64 * (r + 1)])
            pieces.append(mm(s, wint_ref[r]))
            rev_pieces.append(mm(s, wrev_ref[r]))
        fall = jnp.concatenate(pieces, axis=1) * c
        fid = mm(fall, fw[...]) + fb[...]
        fcom = mm(z[:, 320:640] * c, fcw[...]) + fcb[...]
        fdis = mm(z[:, 640:960] * c, fdw[...]) + fdb[...]
        frev = mm(jnp.concatenate(rev_pieces, axis=1) * c, frw[...]) + frb[...]
        return fid, fcom, fdis, frev

    fid_u, fc_u, fd_u, fr_u = side(
        (zu0, zu1, zu2, zu3, zu4, zu5, zu6), zu7, sua, sub, cu,
        fuw, fub, fucw, fucb, fudw, fudb, furw, furb)
    fid_i, fc_i, fd_i, fr_i = side(
        (zi0, zi1, zi2, zi3, zi4, zi5, zi6), zi7, sia, sib, cii,
        fiw, fib, ficw, ficb, fidw, fidb, firw, firb)

    def head(a, b, w1, w2):
        z = a * b
        return mm(jax.nn.relu(mm(z, w1[...])), w2[...])

    oi = head(fid_u, fid_i, piw1, piw2)
    orv = head(fr_u, fr_i, prw1, prw2)
    oc = head(fc_u, fc_i, pcw1, pcw2)
    od = head(fd_u, fd_i, pdw1, pdw2)
    sim = jnp.sum(mm(fc_u, pjw1[...]) * mm(fr_u, pjw2[...]), axis=1,
                  keepdims=True)
    out_ref[...] = jnp.concatenate([oi, orv, oc, od, sim], axis=1)


def _dense_block(u_parts, i_parts, wint, wrev, fcs, preds, proj):
    b = u_parts[0].shape[0]
    blk = 512
    args = list(u_parts) + list(i_parts) + [wint, wrev] + fcs + preds + proj

    def whole(a):
        return pl.BlockSpec(a.shape, lambda i: (0,) * a.ndim)

    def rows(a):
        return pl.BlockSpec((blk, a.shape[1]), lambda i: (i, 0))

    in_specs = ([rows(a) for a in u_parts] + [rows(a) for a in i_parts]
                + [whole(a) for a in args[22:]])
    return pl.pallas_call(
        _dense_body,
        grid=(b // blk,),
        in_specs=in_specs,
        out_specs=pl.BlockSpec((blk, 21), lambda i: (i, 0)),
        out_shape=jax.ShapeDtypeStruct((b, 21), jnp.float32),
    )(*args)


def kernel(edge_index, users, items, ci, review_feat, weight, weight_com,
           weight_dis, review_w_int, review_w_rev,
           fc_user_w, fc_user_b, fc_item_w, fc_item_b,
           fc_user_com_w, fc_user_com_b, fc_item_com_w, fc_item_com_b,
           fc_user_dis_w, fc_user_dis_b, fc_item_dis_w, fc_item_dis_b,
           fc_user_rev_w, fc_user_rev_b, fc_item_rev_w, fc_item_rev_b,
           pred_int_w1, pred_int_w2, pred_rev_w1, pred_rev_w2,
           pred_com_w1, pred_com_w2, pred_dis_w1, pred_dis_w2,
           proj_w1, proj_w2):
    src = edge_index[0]
    dst = edge_index[1]
    iid = items + N_USERS

    ci128 = _build_ci128(ci)
    rfg = _build_rfg_sc(ci128, review_feat, src)
    us = _build_u(weight, weight_com, weight_dis, ci)

    zeros_a = jnp.zeros((640, 128), jnp.float32)
    zs = _segment_sums(src, dst, us, rfg, zeros_a)

    gathered = _row_gathers(list(zs) + [ci128], users, iid)
    u_parts = [gathered[2 * j] for j in range(11)]
    i_parts = [gathered[2 * j + 1] for j in range(11)]

    fcs = [fc_user_w, fc_user_b.reshape(1, -1), fc_item_w,
           fc_item_b.reshape(1, -1),
           fc_user_com_w, fc_user_com_b.reshape(1, -1), fc_item_com_w,
           fc_item_com_b.reshape(1, -1),
           fc_user_dis_w, fc_user_dis_b.reshape(1, -1), fc_item_dis_w,
           fc_item_dis_b.reshape(1, -1),
           fc_user_rev_w, fc_user_rev_b.reshape(1, -1), fc_item_rev_w,
           fc_item_rev_b.reshape(1, -1)]
    preds = [pred_int_w1, pred_int_w2, pred_rev_w1, pred_rev_w2,
             pred_com_w1, pred_com_w2, pred_dis_w1, pred_dis_w2]
    proj = [proj_w1, proj_w2]
    return _dense_block(u_parts, i_parts, review_w_int, review_w_rev, fcs,
                        preds, proj)
